# Initial kernel scaffold; baseline (speedup 1.0000x reference)
import jax
import jax.numpy as jnp
from jax.experimental import pallas as pl


def kernel(pattern_id, pattern_type, form, meaning_class, pattern_table, type_table, form_table, meaning_table):
    raise NotImplementedError("write your pallas kernel here")


# speedup vs baseline: 5.0682x; 5.0682x over previous
import functools

import jax
import jax.numpy as jnp
from jax import lax
from jax.experimental import pallas as pl
from jax.experimental.pallas import tpu as pltpu
from jax.experimental.pallas import tpu_sc as plsc

BATCH = 16384
D = 128
N_TYPE, N_FORM, N_MEAN = 2, 11, 20
N_COMB = N_TYPE * N_FORM * N_MEAN

_info = plsc.get_sparse_core_info()
NC, NS, L = _info.num_cores, _info.num_subcores, _info.num_lanes
NW = NC * NS
BPW = BATCH // NW
K = 128
NCHUNK = BPW // K


def _combine_body(type_ref, form_ref, meaning_ref, out_ref):
    r_t = lax.broadcasted_iota(jnp.int32, (N_COMB, N_TYPE), 0) // (N_FORM * N_MEAN)
    c_t = lax.broadcasted_iota(jnp.int32, (N_COMB, N_TYPE), 1)
    oh_t = jnp.where(c_t == r_t, 1.0, 0.0)
    r_f = (lax.broadcasted_iota(jnp.int32, (N_COMB, N_FORM), 0) // N_MEAN) % N_FORM
    c_f = lax.broadcasted_iota(jnp.int32, (N_COMB, N_FORM), 1)
    oh_f = jnp.where(c_f == r_f, 1.0, 0.0)
    r_m = lax.broadcasted_iota(jnp.int32, (N_COMB, N_MEAN), 0) % N_MEAN
    c_m = lax.broadcasted_iota(jnp.int32, (N_COMB, N_MEAN), 1)
    oh_m = jnp.where(c_m == r_m, 1.0, 0.0)
    out_ref[...] = (
        jnp.dot(oh_t, type_ref[...], preferred_element_type=jnp.float32)
        + jnp.dot(oh_f, form_ref[...], preferred_element_type=jnp.float32)
        + jnp.dot(oh_m, meaning_ref[...], preferred_element_type=jnp.float32)
    )


_combine = pl.pallas_call(
    _combine_body,
    out_shape=jax.ShapeDtypeStruct((N_COMB, D), jnp.float32),
)


def _sc_body(pid_hbm, t_hbm, f_hbm, m_hbm, ptab_hbm, ctab_hbm, out_hbm,
             pid_v, t_v, f_v, m_v, cidx_v, rows_p, rows_c, sem_p, sem_c):
    wid = lax.axis_index("s") * NC + lax.axis_index("c")
    base = wid * BPW
    pltpu.sync_copy(pid_hbm.at[pl.ds(base, BPW)], pid_v)
    pltpu.sync_copy(t_hbm.at[pl.ds(base, BPW)], t_v)
    pltpu.sync_copy(f_hbm.at[pl.ds(base, BPW)], f_v)
    pltpu.sync_copy(m_hbm.at[pl.ds(base, BPW)], m_v)
    for i in range(BPW // L):
        s = pl.ds(i * L, L)
        cidx_v[s] = t_v[s] * (N_FORM * N_MEAN) + f_v[s] * N_MEAN + m_v[s]
    for g in range(NCHUNK):
        cp_p = pltpu.async_copy(
            ptab_hbm.at[pid_v.at[pl.ds(g * K, K)]], rows_p, sem_p)
        cp_c = pltpu.async_copy(
            ctab_hbm.at[cidx_v.at[pl.ds(g * K, K)]], rows_c, sem_c)
        cp_p.wait()
        cp_c.wait()

        def add_row(r, carry):
            for c in range(D // L):
                s = pl.ds(c * L, L)
                rows_p[r, s] = rows_p[r, s] + rows_c[r, s]
            return carry

        lax.fori_loop(0, K, add_row, 0)
        pltpu.sync_copy(rows_p, out_hbm.at[pl.ds(base + g * K, K)])


_sc_gather = functools.partial(
    pl.kernel,
    out_type=jax.ShapeDtypeStruct((BATCH, D), jnp.float32),
    mesh=plsc.VectorSubcoreMesh(core_axis_name="c", subcore_axis_name="s"),
    scratch_types=[
        pltpu.VMEM((BPW,), jnp.int32),
        pltpu.VMEM((BPW,), jnp.int32),
        pltpu.VMEM((BPW,), jnp.int32),
        pltpu.VMEM((BPW,), jnp.int32),
        pltpu.VMEM((BPW,), jnp.int32),
        pltpu.VMEM((K, D), jnp.float32),
        pltpu.VMEM((K, D), jnp.float32),
        pltpu.SemaphoreType.DMA,
        pltpu.SemaphoreType.DMA,
    ],
)(_sc_body)


def kernel(pattern_id, pattern_type, form, meaning_class,
           pattern_table, type_table, form_table, meaning_table):
    pid = pattern_id.astype(jnp.int32)
    t = pattern_type.astype(jnp.int32)
    f = form.astype(jnp.int32)
    m = meaning_class.astype(jnp.int32)
    combined = _combine(type_table, form_table, meaning_table)
    return _sc_gather(pid, t, f, m, pattern_table, combined)
